# baseline (speedup 1.0000x reference)
import functools

import jax
import jax.numpy as jnp
from jax import lax
from jax.experimental import pallas as pl
from jax.experimental.pallas import tpu as pltpu
from jax.experimental.pallas import tpu_sc as plsc

N_CAT = 26
CARD = 100000
N_NUM = 13
D_H = 16
D_M = 64

NC, NS = 2, 16
NW = NC * NS
LANES = 16


def _sc_gather_body(nb, n_chunks, batch, xcat_hbm, table_hbm, out_hbm,
                    xcat_v, idx_v, rows_v, sem):
    wid = lax.axis_index("s") * NC + lax.axis_index("c")
    iota = lax.iota(jnp.int32, LANES)
    for c in range(n_chunks):
        b0 = wid * (nb * n_chunks) + c * nb
        pltpu.sync_copy(xcat_hbm.at[pl.ds(b0 * N_CAT, nb * N_CAT)], xcat_v)
        for j in range(N_CAT):
            off = jnp.int32(j * CARD)

            def body(i, _, j=j, off=off):
                src = (i * LANES + iota) * N_CAT + j
                vals = plsc.load_gather(xcat_v, [src])
                idx_v[pl.ds(j * nb + i * LANES, LANES)] = vals + off
                return 0

            lax.fori_loop(0, nb // LANES, body, 0, unroll=4)
        pltpu.async_copy(table_hbm.at[idx_v], rows_v, sem).wait()
        for j in range(N_CAT):
            pltpu.sync_copy(rows_v.at[pl.ds(j * nb, nb)],
                            out_hbm.at[pl.ds(j * batch + b0, nb)])


def _sc_gather(x_cat, emb_table):
    batch = x_cat.shape[0]
    per_w = batch // NW
    nb = min(256, per_w)
    n_chunks = per_w // nb
    mesh = plsc.VectorSubcoreMesh(core_axis_name="c", subcore_axis_name="s",
                                  num_cores=NC, num_subcores=NS)
    body = functools.partial(_sc_gather_body, nb, n_chunks, batch)
    run = pl.kernel(
        body,
        out_type=jax.ShapeDtypeStruct((N_CAT * batch, D_H), jnp.float32),
        mesh=mesh,
        scratch_types=[
            pltpu.VMEM((nb * N_CAT,), jnp.int32),
            pltpu.VMEM((nb * N_CAT,), jnp.int32),
            pltpu.VMEM((nb * N_CAT, D_H), jnp.float32),
            pltpu.SemaphoreType.DMA,
        ],
        compiler_params=pltpu.CompilerParams(needs_layout_passes=False,
                                             use_tc_tiling_on_sc=False),
    )
    return run(x_cat.reshape(-1), emb_table)


def _tc_body(batch, bb, xnum_ref, nw_ref, nbias_ref, nproj_ref, cb_ref,
             cp_ref, gath_hbm, out_ref, rows_v, sem):
    i = pl.program_id(0)
    g2 = gath_hbm
    copies = []
    for j in range(N_CAT):
        c = pltpu.async_copy(g2.at[pl.ds(j * batch + i * bb, bb)],
                             rows_v.at[pl.ds(j * bb, bb)], sem)
        copies.append(c)

    w2 = jnp.dot(nw_ref[:], nproj_ref[:], preferred_element_type=jnp.float32)
    b2 = jnp.dot(nbias_ref[:], nproj_ref[:], preferred_element_type=jnp.float32)
    bc2 = jnp.dot(cb_ref[:], cp_ref[:], preferred_element_type=jnp.float32)

    xn = xnum_ref[:]
    for k in range(N_NUM):
        col = lax.broadcast_in_dim(xn[:, k], (bb, D_M), (0,))
        w2k = lax.broadcast_in_dim(w2[k], (bb, D_M), (1,))
        b2k = lax.broadcast_in_dim(b2[k], (bb, D_M), (1,))
        out_ref[:, pl.ds(k, 1), :] = (col * w2k + b2k)[:, None, :]

    for c in copies:
        c.wait()
    cp = cp_ref[:]
    for j in range(N_CAT):
        g = rows_v[pl.ds(j * bb, bb), :]
        yj = jnp.dot(g, cp, preferred_element_type=jnp.float32)
        bj = lax.broadcast_in_dim(bc2[j], (bb, D_M), (1,))
        out_ref[:, pl.ds(N_NUM + j, 1), :] = (yj + bj)[:, None, :]


def _tc_fused(x_num, gath, num_weight, num_bias, num_proj, cat_bias, cat_proj):
    batch = x_num.shape[0]
    bb = 512
    grid = (batch // bb,)
    return pl.pallas_call(
        functools.partial(_tc_body, batch, bb),
        grid=grid,
        in_specs=[
            pl.BlockSpec((bb, N_NUM), lambda i: (i, 0)),
            pl.BlockSpec((N_NUM, D_H), lambda i: (0, 0)),
            pl.BlockSpec((N_NUM, D_H), lambda i: (0, 0)),
            pl.BlockSpec((D_H, D_M), lambda i: (0, 0)),
            pl.BlockSpec((N_CAT, D_H), lambda i: (0, 0)),
            pl.BlockSpec((D_H, D_M), lambda i: (0, 0)),
            pl.BlockSpec(memory_space=pl.ANY),
        ],
        out_specs=pl.BlockSpec((bb, N_NUM + N_CAT, D_M), lambda i: (i, 0, 0)),
        out_shape=jax.ShapeDtypeStruct((batch, N_NUM + N_CAT, D_M),
                                       jnp.float32),
        scratch_shapes=[
            pltpu.VMEM((N_CAT * bb, D_H), jnp.float32),
            pltpu.SemaphoreType.DMA,
        ],
    )(x_num, num_weight, num_bias, num_proj, cat_bias, cat_proj, gath)


def kernel(x_num, x_cat, num_weight, num_bias, num_proj, emb_table, cat_bias,
           cat_proj):
    gath = _sc_gather(x_cat, emb_table)
    return _tc_fused(x_num, gath, num_weight, num_bias, num_proj, cat_bias,
                     cat_proj)


# speedup vs baseline: 5.9872x; 5.9872x over previous
import functools

import jax
import jax.numpy as jnp
from jax import lax
from jax.experimental import pallas as pl
from jax.experimental.pallas import tpu as pltpu
from jax.experimental.pallas import tpu_sc as plsc

N_CAT = 26
CARD = 100000
N_NUM = 13
D_H = 16
D_M = 64

NC, NS = 2, 16
NW = NC * NS
LANES = 16
PLANES = N_CAT * D_H


SEG = 100096
GCHUNK = 8192


def _sc_gather_body(batch, xcat_hbm, table_hbm, out_hbm, xv, seg_v, gbuf, sem):
    wid = lax.axis_index("s") * NC + lax.axis_index("c")
    per_w = PLANES // NW
    iota = lax.iota(jnp.int32, LANES)
    del iota
    for q in range(per_w):
        p = wid * per_w + q
        j = p // D_H
        c = p % D_H
        lo = (j * CARD) // 128 * 128
        rel = j * CARD - lo
        pltpu.sync_copy(xcat_hbm.at[j], xv)
        pltpu.async_copy(table_hbm.at[c].at[pl.ds(lo, SEG)], seg_v, sem).wait()
        for h in range(batch // GCHUNK):

            def body(i, _, h=h):
                sl = pl.ds(h * GCHUNK + i * LANES, LANES)
                v = xv[sl] + rel
                gbuf[pl.ds(i * LANES, LANES)] = plsc.load_gather(seg_v, [v])
                return 0

            lax.fori_loop(0, GCHUNK // LANES, body, 0, unroll=8)
            pltpu.sync_copy(gbuf, out_hbm.at[j, c, pl.ds(h * GCHUNK, GCHUNK)])


def _sc_gather(x_cat_t, emb_table_t):
    batch = x_cat_t.shape[1]
    mesh = plsc.VectorSubcoreMesh(core_axis_name="c", subcore_axis_name="s",
                                  num_cores=NC, num_subcores=NS)
    body = functools.partial(_sc_gather_body, batch)
    run = pl.kernel(
        body,
        out_type=jax.ShapeDtypeStruct((N_CAT, D_H, batch), jnp.float32),
        mesh=mesh,
        scratch_types=[
            pltpu.VMEM((batch,), jnp.int32),
            pltpu.VMEM((SEG,), jnp.float32),
            pltpu.VMEM((GCHUNK,), jnp.float32),
            pltpu.SemaphoreType.DMA,
        ],
        compiler_params=pltpu.CompilerParams(needs_layout_passes=False,
                                             use_tc_tiling_on_sc=True,
                                             disable_bounds_checks=True),
    )
    return run(x_cat_t, emb_table_t)


def _tc_body(xnum_ref, nw_ref, nbias_ref, nproj_ref, cbt_ref, cp_ref,
             gath_ref, out_ref):
    bb = xnum_ref.shape[1]
    npT = jnp.transpose(nproj_ref[:])
    w2T = jnp.dot(npT, jnp.transpose(nw_ref[:]),
                  preferred_element_type=jnp.float32)
    b2T = jnp.dot(npT, jnp.transpose(nbias_ref[:]),
                  preferred_element_type=jnp.float32)
    cpT = jnp.transpose(cp_ref[:])
    bcT = jnp.dot(cpT, cbt_ref[:],
                  preferred_element_type=jnp.float32)

    for k in range(N_NUM):
        xk = lax.broadcast_in_dim(xnum_ref[k], (D_M, bb), (1,))
        wk = lax.broadcast_in_dim(w2T[:, k], (D_M, bb), (0,))
        bk = lax.broadcast_in_dim(b2T[:, k], (D_M, bb), (0,))
        out_ref[pl.ds(k, 1)] = (xk * wk + bk)[None]

    for j in range(N_CAT):
        gj = gath_ref[j]
        y = jnp.dot(cpT, gj, preferred_element_type=jnp.float32)
        bj = lax.broadcast_in_dim(bcT[:, j], (D_M, bb), (0,))
        out_ref[pl.ds(N_NUM + j, 1)] = (y + bj)[None]


def _tc_fused(x_num_t, gath, num_weight, num_bias, num_proj, cat_bias_t,
              cat_proj):
    batch = x_num_t.shape[1]
    bb = 512
    grid = (batch // bb,)
    n_out = N_NUM + N_CAT
    return pl.pallas_call(
        _tc_body,
        grid=grid,
        in_specs=[
            pl.BlockSpec((N_NUM, bb), lambda i: (0, i)),
            pl.BlockSpec((N_NUM, D_H), lambda i: (0, 0)),
            pl.BlockSpec((N_NUM, D_H), lambda i: (0, 0)),
            pl.BlockSpec((D_H, D_M), lambda i: (0, 0)),
            pl.BlockSpec((D_H, N_CAT), lambda i: (0, 0)),
            pl.BlockSpec((D_H, D_M), lambda i: (0, 0)),
            pl.BlockSpec((N_CAT, D_H, bb), lambda i: (0, 0, i)),
        ],
        out_specs=pl.BlockSpec((n_out, D_M, bb), lambda i: (0, 0, i)),
        out_shape=jax.ShapeDtypeStruct((n_out, D_M, batch), jnp.float32),
    )(x_num_t, num_weight, num_bias, num_proj, cat_bias_t, cat_proj, gath)


def kernel(x_num, x_cat, num_weight, num_bias, num_proj, emb_table, cat_bias,
           cat_proj):
    gath = _sc_gather(x_cat.T, emb_table.T)
    out_t = _tc_fused(x_num.T, gath, num_weight, num_bias, num_proj,
                      cat_bias.T, cat_proj)
    return jnp.transpose(out_t, (2, 0, 1))
